# baseline (speedup 1.0000x reference)
import jax
import jax.numpy as jnp
from jax import lax
from jax.experimental import pallas as pl
from jax.experimental.pallas import tpu as pltpu

B, T, H, D = 2, 256, 8, 256
E, K, FFN = 8, 2, 1024
OUT = 5 * D
N = B * T * H
BLK = 1024
EMB_N = T * H

_F32 = jnp.float32
_BF16 = jnp.bfloat16
_INV_SQRT2 = 0.7071067811865476


def _moe_body(x_ref, emb_ref, wg_ref, w1_hbm, b1_ref, w2_hbm, b2_ref,
              wp_ref, bp_ref, out_ref,
              h_s, w1f_s, w2f_s, w1b_s, w2b_s, wpb_s,
              sem1, sem2):
    i = pl.program_id(0)

    @pl.when(i == 0)
    def _start_dma():
        pltpu.make_async_copy(w1_hbm, w1f_s, sem1).start()
        pltpu.make_async_copy(w2_hbm, w2f_s, sem2).start()

    x32 = x_ref[...] + emb_ref[...]
    logits = jnp.dot(x32, wg_ref[...], preferred_element_type=_F32)
    gates = jax.nn.softmax(logits, axis=-1)
    eidx = lax.broadcasted_iota(jnp.int32, (BLK, E), 1)
    i1 = jnp.argmax(gates, axis=1)
    oh1 = (eidx == i1[:, None])
    v1 = jnp.max(gates, axis=1)
    g2 = jnp.where(oh1, -jnp.inf, gates)
    i2 = jnp.argmax(g2, axis=1)
    oh2 = (eidx == i2[:, None])
    v2 = jnp.max(g2, axis=1)
    s = v1 + v2
    mask = (oh1 * (v1 / s)[:, None] + oh2 * (v2 / s)[:, None]).astype(_F32)
    maskb = mask.astype(_BF16)

    @pl.when(i == 0)
    def _cvt_w1():
        pltpu.make_async_copy(w1_hbm, w1f_s, sem1).wait()
        w1b_s[...] = w1f_s[...].astype(_BF16)

    xb = x32.astype(_BF16)
    acc = jnp.dot(mask, b2_ref[...], preferred_element_type=_F32)
    for half in range(2):
        for j in range(E // 2):
            e = half * (E // 2) + j
            h = jnp.dot(xb, w1b_s[e],
                        preferred_element_type=_F32).astype(_BF16)
            h = h + b1_ref[e][None, :]
            g = h * (0.5 * (1.0 + lax.erf(h * _INV_SQRT2)))
            h_s[:, j * FFN:(j + 1) * FFN] = g * maskb[:, e][:, None]
        if half == 0:
            @pl.when(i == 0)
            def _cvt_w2():
                pltpu.make_async_copy(w2_hbm, w2f_s, sem2).wait()
                w2b_s[...] = w2f_s[...].astype(_BF16)
                wpb_s[...] = wp_ref[...].astype(_BF16)
        lo = half * (E // 2) * FFN
        acc = acc + jnp.dot(h_s[...], w2b_s[lo:lo + (E // 2) * FFN],
                            preferred_element_type=_F32)
    y = jnp.dot(jax.nn.gelu(acc).astype(_BF16), wpb_s[...],
                preferred_element_type=_F32) + bp_ref[...]
    out_ref[...] = y


@jax.jit
def kernel(x, embedding, Wg, W1, b1, W2, b2, Wp, bp):
    xt = x.reshape(N, D)
    emb = embedding.reshape(EMB_N, D)
    nb_e = EMB_N // BLK
    out = pl.pallas_call(
        _moe_body,
        grid=(N // BLK,),
        in_specs=[
            pl.BlockSpec((BLK, D), lambda i: (i, 0)),
            pl.BlockSpec((BLK, D), lambda i: (lax.rem(i, nb_e), 0)),
            pl.BlockSpec((D, E), lambda i: (0, 0)),
            pl.BlockSpec(memory_space=pl.ANY),
            pl.BlockSpec((E, FFN), lambda i: (0, 0)),
            pl.BlockSpec(memory_space=pl.ANY),
            pl.BlockSpec((E, D), lambda i: (0, 0)),
            pl.BlockSpec((D, OUT), lambda i: (0, 0)),
            pl.BlockSpec((1, OUT), lambda i: (0, 0)),
        ],
        out_specs=pl.BlockSpec((BLK, OUT), lambda i: (i, 0)),
        out_shape=jax.ShapeDtypeStruct((N, OUT), _F32),
        scratch_shapes=[
            pltpu.VMEM((BLK, E // 2 * FFN), _BF16),
            pltpu.VMEM((E, D, FFN), _F32),
            pltpu.VMEM((E * FFN, D), _F32),
            pltpu.VMEM((E, D, FFN), _BF16),
            pltpu.VMEM((E * FFN, D), _BF16),
            pltpu.VMEM((D, OUT), _BF16),
            pltpu.SemaphoreType.DMA,
            pltpu.SemaphoreType.DMA,
        ],
        compiler_params=pltpu.CompilerParams(
            dimension_semantics=("arbitrary",)),
    )(xt, emb, Wg, W1, b1.astype(_BF16), W2.reshape(E * FFN, D), b2,
      Wp, bp.reshape(1, OUT))
    return out.reshape(B, T, H, OUT)


# speedup vs baseline: 1.2069x; 1.2069x over previous
import jax
import jax.numpy as jnp
from jax import lax
from jax.experimental import pallas as pl
from jax.experimental.pallas import tpu as pltpu

B, T, H, D = 2, 256, 8, 256
E, K, FFN = 8, 2, 1024
OUT = 5 * D
N = B * T * H
BLK = 1024
EMB_N = T * H

_F32 = jnp.float32
_BF16 = jnp.bfloat16
_INV_SQRT2 = 0.7071067811865476


def _moe_body(x_ref, emb_ref, wg_ref, w1_ref, b1_ref, w2_ref, b2_ref,
              wp_ref, bp_ref, out_ref, h_s, w1b_s, w2b_s, wpb_s):
    @pl.when(pl.program_id(0) == 0)
    def _cvt():
        w1b_s[...] = w1_ref[...].astype(_BF16)
        w2b_s[...] = w2_ref[...].astype(_BF16)
        wpb_s[...] = wp_ref[...].astype(_BF16)

    x32 = x_ref[...] + emb_ref[...]
    logits = jnp.dot(x32, wg_ref[...], preferred_element_type=_F32)
    gates = jax.nn.softmax(logits, axis=-1)
    eidx = lax.broadcasted_iota(jnp.int32, (BLK, E), 1)
    i1 = jnp.argmax(gates, axis=1)
    oh1 = (eidx == i1[:, None])
    v1 = jnp.max(gates, axis=1)
    g2 = jnp.where(oh1, -jnp.inf, gates)
    i2 = jnp.argmax(g2, axis=1)
    oh2 = (eidx == i2[:, None])
    v2 = jnp.max(g2, axis=1)
    s = v1 + v2
    mask = (oh1 * (v1 / s)[:, None] + oh2 * (v2 / s)[:, None]).astype(_F32)
    maskb = mask.astype(_BF16)

    xb = x32.astype(_BF16)
    for e in range(E):
        h = jnp.dot(xb, w1b_s[e],
                    preferred_element_type=_F32).astype(_BF16)
        h = h + b1_ref[e][None, :]
        g = h * (0.5 * (1.0 + lax.erf(h * _INV_SQRT2)))
        h_s[:, e * FFN:(e + 1) * FFN] = g * maskb[:, e][:, None]
    acc = jnp.dot(h_s[...], w2b_s[...], preferred_element_type=_F32)
    acc = acc + jnp.dot(mask, b2_ref[...], preferred_element_type=_F32)
    y = jnp.dot(jax.nn.gelu(acc).astype(_BF16), wpb_s[...],
                preferred_element_type=_F32) + bp_ref[...]
    out_ref[...] = y


@jax.jit
def kernel(x, embedding, Wg, W1, b1, W2, b2, Wp, bp):
    xt = x.reshape(N, D)
    emb = embedding.reshape(EMB_N, D)
    nb_e = EMB_N // BLK
    out = pl.pallas_call(
        _moe_body,
        grid=(N // BLK,),
        in_specs=[
            pl.BlockSpec((BLK, D), lambda i: (i, 0)),
            pl.BlockSpec((BLK, D), lambda i: (lax.rem(i, nb_e), 0)),
            pl.BlockSpec((D, E), lambda i: (0, 0)),
            pl.BlockSpec((E, D, FFN), lambda i: (0, 0, 0)),
            pl.BlockSpec((E, FFN), lambda i: (0, 0)),
            pl.BlockSpec((E * FFN, D), lambda i: (0, 0)),
            pl.BlockSpec((E, D), lambda i: (0, 0)),
            pl.BlockSpec((D, OUT), lambda i: (0, 0)),
            pl.BlockSpec((1, OUT), lambda i: (0, 0)),
        ],
        out_specs=pl.BlockSpec((BLK, OUT), lambda i: (i, 0)),
        out_shape=jax.ShapeDtypeStruct((N, OUT), _F32),
        scratch_shapes=[
            pltpu.VMEM((BLK, E * FFN), _BF16),
            pltpu.VMEM((E, D, FFN), _BF16),
            pltpu.VMEM((E * FFN, D), _BF16),
            pltpu.VMEM((D, OUT), _BF16),
        ],
        compiler_params=pltpu.CompilerParams(
            dimension_semantics=("arbitrary",)),
    )(xt, emb, Wg, W1, b1.astype(_BF16), W2.reshape(E * FFN, D), b2,
      Wp, bp.reshape(1, OUT))
    return out.reshape(B, T, H, OUT)
